# baseline (speedup 1.0000x reference)
import jax
import jax.numpy as jnp
from jax import lax
from jax.experimental import pallas as pl
from jax.experimental.pallas import tpu as pltpu
from jax.experimental.pallas import tpu_sc as plsc

_D = 64
_SEQ = 200
_BATCH = 4096
_SCALE = float(_D) ** 0.5

_NC = 2
_NS = 16
_NW = _NC * _NS
_SEQ_PER_W = _BATCH // _NW
_LANES = 16
_VPR = _D // _LANES


def _sc_body(x_hbm, table_hbm, pe_hbm, out_hbm, pe_v, idx_v, rows_v, sem):
    wid = lax.axis_index("s") * _NC + lax.axis_index("c")
    base = wid * _SEQ_PER_W

    pltpu.sync_copy(pe_hbm, pe_v)

    def seq_body(b, carry):
        flat = (base + b) * _SEQ
        pltpu.sync_copy(x_hbm.at[pl.ds(flat, _SEQ)], idx_v)
        pltpu.async_copy(table_hbm.at[idx_v], rows_v, sem).wait()

        def row_body(r, c):
            for j in range(_VPR):
                sl = pl.ds(j * _LANES, _LANES)
                rows_v[r, sl] = rows_v[r, sl] * _SCALE + pe_v[r, sl]
            return c

        lax.fori_loop(0, _SEQ, row_body, 0, unroll=2)
        pltpu.sync_copy(rows_v, out_hbm.at[pl.ds(flat, _SEQ)])
        return carry

    lax.fori_loop(0, _SEQ_PER_W, seq_body, 0)


@jax.jit
def _embed(x_flat, table, pe_seq):
    mesh = plsc.VectorSubcoreMesh(core_axis_name="c", subcore_axis_name="s")
    launch = pl.kernel(
        _sc_body,
        out_type=jax.ShapeDtypeStruct((_BATCH * _SEQ, _D), jnp.float32),
        mesh=mesh,
        scratch_types=[
            pltpu.VMEM((_SEQ, _D), jnp.float32),
            pltpu.VMEM((_SEQ,), jnp.int32),
            pltpu.VMEM((_SEQ, _D), jnp.float32),
            pltpu.SemaphoreType.DMA,
        ],
        compiler_params=pltpu.CompilerParams(use_tc_tiling_on_sc=False),
    )
    return launch(x_flat, table, pe_seq)


def kernel(x, table, pe):
    x_flat = x.reshape(-1).astype(jnp.int32)
    pe_seq = pe[: x.shape[1]].astype(jnp.float32)
    out = _embed(x_flat, table, pe_seq)
    return out.reshape(x.shape[0], x.shape[1], _D)


# speedup vs baseline: 1.1759x; 1.1759x over previous
import jax
import jax.numpy as jnp
from jax import lax
from jax.experimental import pallas as pl
from jax.experimental.pallas import tpu as pltpu
from jax.experimental.pallas import tpu_sc as plsc

_D = 64
_SEQ = 200
_BATCH = 4096
_SCALE = float(_D) ** 0.5

_NC = 2
_NS = 16
_NW = _NC * _NS
_SEQ_PER_W = _BATCH // _NW
_LANES = 16
_VPR = _D // _LANES

_NBUF = 4
_LOOK = _NBUF - 1
_NCHUNK = _SEQ_PER_W


def _sc_body(x_hbm, table_hbm, pe_hbm, out_hbm, pe_v, idx_v,
             r0, r1, r2, r3, g0, g1, g2, g3, s0, s1, s2, s3):
    rows = [r0, r1, r2, r3]
    gsem = [g0, g1, g2, g3]
    ssem = [s0, s1, s2, s3]
    wid = lax.axis_index("s") * _NC + lax.axis_index("c")
    seq0 = wid * _SEQ_PER_W

    pltpu.sync_copy(pe_hbm, pe_v)
    pltpu.sync_copy(x_hbm.at[pl.ds(seq0 * _SEQ, _SEQ_PER_W * _SEQ)], idx_v)

    def g_start(c, b):
        pltpu.async_copy(table_hbm.at[idx_v.at[pl.ds(c * _SEQ, _SEQ)]],
                         rows[b], gsem[b])

    def g_wait(c, b):
        pltpu.make_async_copy(table_hbm.at[idx_v.at[pl.ds(c * _SEQ, _SEQ)]],
                              rows[b], gsem[b]).wait()

    def s_start(c, b):
        pltpu.async_copy(rows[b], out_hbm.at[pl.ds((seq0 + c) * _SEQ, _SEQ)],
                         ssem[b])

    def s_wait(c, b):
        pltpu.make_async_copy(rows[b],
                              out_hbm.at[pl.ds((seq0 + c) * _SEQ, _SEQ)],
                              ssem[b]).wait()

    for c in range(_LOOK):
        g_start(c, c)

    def slot(c, b):
        g_wait(c, b)

        def row_body(r, carry):
            for j in range(_VPR):
                sl = pl.ds(j * _LANES, _LANES)
                rows[b][r, sl] = rows[b][r, sl] * _SCALE + pe_v[r, sl]
            return carry

        lax.fori_loop(0, _SEQ, row_body, 0, unroll=2)
        s_start(c, b)

        cn = c + _LOOK
        bn = (b + _LOOK) % _NBUF

        @pl.when(cn < _NCHUNK)
        def _():
            @pl.when(cn >= _NBUF)
            def _():
                s_wait(cn - _NBUF, bn)
            g_start(cn, bn)

    def group(i, carry):
        for b in range(_NBUF):
            slot(i * _NBUF + b, b)
        return carry

    lax.fori_loop(0, _NCHUNK // _NBUF, group, 0)

    for b in range(_NBUF):
        s_wait(_NCHUNK - _NBUF + b, b)


@jax.jit
def _embed(x_flat, table, pe_seq):
    mesh = plsc.VectorSubcoreMesh(core_axis_name="c", subcore_axis_name="s")
    launch = pl.kernel(
        _sc_body,
        out_type=jax.ShapeDtypeStruct((_BATCH * _SEQ, _D), jnp.float32),
        mesh=mesh,
        scratch_types=(
            [pltpu.VMEM((_SEQ, _D), jnp.float32)]
            + [pltpu.VMEM((_SEQ_PER_W * _SEQ,), jnp.int32)]
            + [pltpu.VMEM((_SEQ, _D), jnp.float32)] * _NBUF
            + [pltpu.SemaphoreType.DMA] * (2 * _NBUF)
        ),
        compiler_params=pltpu.CompilerParams(use_tc_tiling_on_sc=False),
    )
    return launch(x_flat, table, pe_seq)


def kernel(x, table, pe):
    x_flat = x.reshape(-1).astype(jnp.int32)
    pe_seq = pe[: x.shape[1]].astype(jnp.float32)
    out = _embed(x_flat, table, pe_seq)
    return out.reshape(x.shape[0], x.shape[1], _D)
